# Initial kernel scaffold; baseline (speedup 1.0000x reference)
import jax
import jax.numpy as jnp
from jax.experimental import pallas as pl


def kernel(x, embeddings, nodes, graph, clusters, Wn1, bn1, Wn2, bn2, We1, be1, We2, be2):
    raise NotImplementedError("write your pallas kernel here")


# speedup vs baseline: 5.5710x; 5.5710x over previous
import functools

import jax
import jax.numpy as jnp
from jax import lax
from jax.experimental import pallas as pl

N = 10000
LAT = 128
EMB = 16
NC = 500
NCP = 512
HID = 256
K_SUP = 8
K_BIP = 4
EPS = 1e-12
BLK = 1000
NEG = -3.0e38


def _ln(h):
    m = h.mean(-1, keepdims=True)
    v = ((h - m) ** 2).mean(-1, keepdims=True)
    return (h - m) * lax.rsqrt(v + 1e-5)


def _topk_iter(work, iota_c, k):
    R, C = work.shape
    mask = jnp.zeros((R, C), dtype=jnp.float32)
    vals, idxs, sels = [], [], []
    for _ in range(k):
        mx = jnp.max(work, axis=1, keepdims=True)
        ismax = work == mx
        j = jnp.min(jnp.where(ismax, iota_c, C), axis=1, keepdims=True)
        sel = iota_c == j
        mask = jnp.where(sel, 1.0, mask)
        work = jnp.where(sel, NEG, work)
        vals.append(mx)
        idxs.append(j)
        sels.append(sel)
    return mask, vals, idxs, sels


def _k1_body(emb_ref, cl_ref, sums_ref, cnt_ref):
    i = pl.program_id(0)
    emb = emb_ref[...]
    nrm = jnp.sqrt(jnp.sum(emb * emb, axis=1, keepdims=True))
    embn = emb / (nrm + EPS)
    cl = cl_ref[...]
    iota_c = lax.broadcasted_iota(jnp.int32, (1, NCP), 1)
    onehot = (cl == iota_c).astype(jnp.float32)
    sums = lax.dot_general(onehot, embn, (((0,), (0,)), ((), ())),
                           preferred_element_type=jnp.float32, precision=lax.Precision.HIGHEST)
    ones = jnp.ones((BLK, 8), dtype=jnp.float32)
    cnts = lax.dot_general(onehot, ones, (((0,), (0,)), ((), ())),
                           preferred_element_type=jnp.float32, precision=lax.Precision.HIGHEST)

    @pl.when(i == 0)
    def _():
        sums_ref[...] = jnp.zeros_like(sums_ref)
        cnt_ref[...] = jnp.zeros_like(cnt_ref)

    sums_ref[...] += sums
    cnt_ref[...] += cnts


def _run_k1(embeddings, clusters2d):
    grid = N // BLK
    return pl.pallas_call(
        _k1_body,
        grid=(grid,),
        in_specs=[
            pl.BlockSpec((BLK, EMB), lambda i: (i, 0)),
            pl.BlockSpec((BLK, 1), lambda i: (i, 0)),
        ],
        out_specs=[
            pl.BlockSpec((NCP, EMB), lambda i: (0, 0)),
            pl.BlockSpec((NCP, 8), lambda i: (0, 0)),
        ],
        out_shape=[
            jax.ShapeDtypeStruct((NCP, EMB), jnp.float32),
            jax.ShapeDtypeStruct((NCP, 8), jnp.float32),
        ],
    )(embeddings, clusters2d)


def _k2_body(sums_ref, cnt_ref, means_ref, idx_ref, wf_ref, ws_ref):
    sums = sums_ref[...]
    cnt = cnt_ref[:, 0:1]
    mean = sums / jnp.maximum(cnt, 1.0)
    nrm = jnp.sqrt(jnp.sum(mean * mean, axis=1, keepdims=True))
    means = mean / (nrm + EPS)
    means_ref[...] = means

    sim_d = lax.dot_general(means, means, (((1,), (1,)), ((), ())),
                            preferred_element_type=jnp.float32)
    sim = lax.dot_general(means, means, (((1,), (1,)), ((), ())),
                          preferred_element_type=jnp.float32, precision=lax.Precision.HIGHEST)
    iota_c = lax.broadcasted_iota(jnp.int32, (1, NCP), 1)
    row_i = lax.broadcasted_iota(jnp.int32, (NCP, 1), 0)
    valid_col = iota_c < NC
    work = jnp.where(valid_col, sim_d, NEG)
    mask, _, idxs, sels = _topk_iter(work, iota_c, K_SUP)

    y = jnp.clip(sim, -1 + 1e-7, 1 - 1e-7)
    s = jnp.sqrt((1 + y) / (1 - y))
    wmat = s / (1 + s)
    a = mask * wmat
    a = jnp.where(row_i < NC, a, 0.0)
    onescol = jnp.ones((NCP, 1), dtype=jnp.float32)
    colsum = lax.dot_general(a, onescol, (((0,), (0,)), ((), ())),
                             preferred_element_type=jnp.float32, precision=lax.Precision.HIGHEST)
    rowsum = lax.dot_general(a, onescol, (((1,), (0,)), ((), ())),
                             preferred_element_type=jnp.float32, precision=lax.Precision.HIGHEST)
    denom = colsum + rowsum

    for t in range(K_SUP):
        j = idxs[t]
        mx = jnp.sum(jnp.where(sels[t], sim, 0.0), axis=1, keepdims=True)
        yv = jnp.clip(mx, -1 + 1e-7, 1 - 1e-7)
        sv = jnp.sqrt((1 + yv) / (1 - yv))
        wv = sv / (1 + sv)
        oh = (iota_c == j).astype(jnp.float32)
        dg = lax.dot_general(oh, denom, (((1,), (0,)), ((), ())),
                             preferred_element_type=jnp.float32, precision=lax.Precision.HIGHEST)
        idx_ref[:, t:t + 1] = j
        wf_ref[:, t:t + 1] = wv / (dg + EPS)
        ws_ref[:, t:t + 1] = wv / (denom + EPS)


def _run_k2(sums, cnts):
    return pl.pallas_call(
        _k2_body,
        out_shape=[
            jax.ShapeDtypeStruct((NCP, EMB), jnp.float32),
            jax.ShapeDtypeStruct((NCP, K_SUP), jnp.int32),
            jax.ShapeDtypeStruct((NCP, K_SUP), jnp.float32),
            jax.ShapeDtypeStruct((NCP, K_SUP), jnp.float32),
        ],
    )(sums, cnts)


def _k3_body(emb_ref, nodes_ref, means_ref, sraw_ref, den_ref):
    i = pl.program_id(0)
    emb = emb_ref[...]
    nrm = jnp.sqrt(jnp.sum(emb * emb, axis=1, keepdims=True))
    embn = emb / (nrm + EPS)
    means = means_ref[...]
    bsim = lax.dot_general(embn, means, (((1,), (1,)), ((), ())),
                           preferred_element_type=jnp.float32)
    iota_c = lax.broadcasted_iota(jnp.int32, (1, NCP), 1)
    work = jnp.where(iota_c < NC, bsim, NEG)
    mask, _, _, _ = _topk_iter(work, iota_c, K_BIP)

    y = jnp.clip(bsim, -1 + 1e-7, 1 - 1e-7)
    w = mask * jnp.sqrt((1 + y) / (1 - y))

    nodes = nodes_ref[...]
    l1 = jnp.sum(jnp.abs(nodes), axis=1, keepdims=True)
    nodes_l1 = nodes / (l1 + EPS)

    sraw = lax.dot_general(w, nodes_l1, (((0,), (0,)), ((), ())),
                           preferred_element_type=jnp.float32, precision=lax.Precision.HIGHEST)
    ones = jnp.ones((BLK, 8), dtype=jnp.float32)
    den = lax.dot_general(w, ones, (((0,), (0,)), ((), ())),
                          preferred_element_type=jnp.float32, precision=lax.Precision.HIGHEST)

    @pl.when(i == 0)
    def _():
        sraw_ref[...] = jnp.zeros_like(sraw_ref)
        den_ref[...] = jnp.zeros_like(den_ref)

    sraw_ref[...] += sraw
    den_ref[...] += den


def _run_k3(embeddings, nodes, means):
    grid = N // BLK
    return pl.pallas_call(
        _k3_body,
        grid=(grid,),
        in_specs=[
            pl.BlockSpec((BLK, EMB), lambda i: (i, 0)),
            pl.BlockSpec((BLK, LAT), lambda i: (i, 0)),
            pl.BlockSpec((NCP, EMB), lambda i: (0, 0)),
        ],
        out_specs=[
            pl.BlockSpec((NCP, LAT), lambda i: (0, 0)),
            pl.BlockSpec((NCP, 8), lambda i: (0, 0)),
        ],
        out_shape=[
            jax.ShapeDtypeStruct((NCP, LAT), jnp.float32),
            jax.ShapeDtypeStruct((NCP, 8), jnp.float32),
        ],
    )(embeddings, nodes, means)


def _k4_body(sraw_ref, den_ref, means_ref, idxf_ref,
             wn1_ref, bn1_ref, wn2_ref, bn2_ref,
             w1at_ref, w1ab_ref, w1bt_ref, w1bb_ref, be1_ref,
             we2_ref, be2_ref,
             snenc_ref, sef_ref, ses_ref):
    sn = sraw_ref[...] / (den_ref[:, 0:1] + EPS)
    h = jax.nn.relu(_ln(jnp.dot(sn, wn1_ref[...],
                                preferred_element_type=jnp.float32, precision=lax.Precision.HIGHEST)
                        + bn1_ref[...]))
    sn_enc = jax.nn.relu(_ln(jnp.dot(h, wn2_ref[...],
                                     preferred_element_type=jnp.float32, precision=lax.Precision.HIGHEST)
                             + bn2_ref[...]))
    snenc_ref[...] = sn_enc

    means = means_ref[...]
    p = (jnp.dot(means, w1at_ref[...], preferred_element_type=jnp.float32, precision=lax.Precision.HIGHEST)
         + jnp.dot(sn_enc, w1ab_ref[...], preferred_element_type=jnp.float32, precision=lax.Precision.HIGHEST))
    q = (jnp.dot(means, w1bt_ref[...], preferred_element_type=jnp.float32, precision=lax.Precision.HIGHEST)
         + jnp.dot(sn_enc, w1bb_ref[...], preferred_element_type=jnp.float32, precision=lax.Precision.HIGHEST))

    idxf = idxf_ref[...]
    iota_c = lax.broadcasted_iota(jnp.int32, (1, NCP), 1)
    ohe = (idxf == iota_c).astype(jnp.float32)
    pg = jnp.dot(ohe, p, preferred_element_type=jnp.float32, precision=lax.Precision.HIGHEST)
    qg = jnp.dot(ohe, q, preferred_element_type=jnp.float32, precision=lax.Precision.HIGHEST)
    p_rep = jnp.reshape(jnp.broadcast_to(p[:, None, :], (NCP, K_SUP, HID)),
                        (NCP * K_SUP, HID))
    q_rep = jnp.reshape(jnp.broadcast_to(q[:, None, :], (NCP, K_SUP, HID)),
                        (NCP * K_SUP, HID))

    be1 = be1_ref[...]
    we2 = we2_ref[...]
    be2 = be2_ref[...]
    h1f = jax.nn.relu(_ln(p_rep + qg + be1))
    h1s = jax.nn.relu(_ln(pg + q_rep + be1))
    sef_ref[...] = jax.nn.relu(_ln(jnp.dot(h1f, we2,
                                           preferred_element_type=jnp.float32, precision=lax.Precision.HIGHEST)
                                   + be2))
    ses_ref[...] = jax.nn.relu(_ln(jnp.dot(h1s, we2,
                                           preferred_element_type=jnp.float32, precision=lax.Precision.HIGHEST)
                                   + be2))


def _run_k4(sraw, den, means, idxf, wn1, bn1, wn2, bn2,
            w1at, w1ab, w1bt, w1bb, be1, we2, be2):
    ne = NCP * K_SUP
    return pl.pallas_call(
        _k4_body,
        out_shape=[
            jax.ShapeDtypeStruct((NCP, LAT - EMB), jnp.float32),
            jax.ShapeDtypeStruct((ne, LAT), jnp.float32),
            jax.ShapeDtypeStruct((ne, LAT), jnp.float32),
        ],
    )(sraw, den, means, idxf, wn1, bn1, wn2, bn2,
      w1at, w1ab, w1bt, w1bb, be1, we2, be2)


@jax.jit
def kernel(x, embeddings, nodes, graph, clusters,
           Wn1, bn1, Wn2, bn2, We1, be1, We2, be2):
    del x, graph
    clusters2d = clusters.reshape(N, 1).astype(jnp.int32)
    sums, cnts = _run_k1(embeddings, clusters2d)
    means, idx, wf, ws = _run_k2(sums, cnts)
    sraw, den = _run_k3(embeddings, nodes, means)

    idxf = idx.reshape(NCP * K_SUP, 1)
    w1at, w1ab = We1[:EMB], We1[EMB:LAT]
    w1bt, w1bb = We1[LAT:LAT + EMB], We1[LAT + EMB:]
    sn_enc, sef, ses = _run_k4(
        sraw, den, means, idxf, Wn1, bn1.reshape(1, HID),
        Wn2, bn2.reshape(1, LAT - EMB),
        w1at, w1ab, w1bt, w1bb, be1.reshape(1, HID),
        We2, be2.reshape(1, LAT))

    supernodes = jnp.concatenate([means[:NC], sn_enc[:NC]], axis=-1)
    nedge = NC * K_SUP
    superedges = jnp.concatenate([sef[:nedge], ses[:nedge]], axis=0)
    sew = jnp.concatenate([wf.reshape(-1)[:nedge], ws.reshape(-1)[:nedge]])
    return supernodes, superedges, sew[:, None]


# speedup vs baseline: 7.8519x; 1.4094x over previous
import functools

import jax
import jax.numpy as jnp
from jax import lax
from jax.experimental import pallas as pl

N = 10000
LAT = 128
EMB = 16
NC = 500
NCP = 512
HID = 256
K_SUP = 8
K_BIP = 4
EPS = 1e-12
BLK = 1000
NEG = -3.0e38


def _bf16_split(x):
    hi = x.astype(jnp.bfloat16).astype(jnp.float32)
    return hi, x - hi


def _dot_exact_lhs(a, b, dims):
    b_hi, b_lo = _bf16_split(b)
    d1 = lax.dot_general(a, b_hi, dims, preferred_element_type=jnp.float32)
    d2 = lax.dot_general(a, b_lo, dims, preferred_element_type=jnp.float32)
    return d1 + d2


def _dot_exact_rhs(a, b, dims):
    a_hi, a_lo = _bf16_split(a)
    d1 = lax.dot_general(a_hi, b, dims, preferred_element_type=jnp.float32)
    d2 = lax.dot_general(a_lo, b, dims, preferred_element_type=jnp.float32)
    return d1 + d2


def _dot3(a, b, dims):
    a_hi, a_lo = _bf16_split(a)
    b_hi, b_lo = _bf16_split(b)
    d1 = lax.dot_general(a_hi, b_hi, dims, preferred_element_type=jnp.float32)
    d2 = lax.dot_general(a_hi, b_lo, dims, preferred_element_type=jnp.float32)
    d3 = lax.dot_general(a_lo, b_hi, dims, preferred_element_type=jnp.float32)
    return d1 + (d2 + d3)


def _ln(h):
    m = h.mean(-1, keepdims=True)
    v = ((h - m) ** 2).mean(-1, keepdims=True)
    return (h - m) * lax.rsqrt(v + 1e-5)


def _topk_iter(work, iota_c, k):
    R, C = work.shape
    mask = jnp.zeros((R, C), dtype=jnp.float32)
    vals, idxs, sels = [], [], []
    for _ in range(k):
        mx = jnp.max(work, axis=1, keepdims=True)
        ismax = work == mx
        j = jnp.min(jnp.where(ismax, iota_c, C), axis=1, keepdims=True)
        sel = iota_c == j
        mask = jnp.where(sel, 1.0, mask)
        work = jnp.where(sel, NEG, work)
        vals.append(mx)
        idxs.append(j)
        sels.append(sel)
    return mask, vals, idxs, sels


def _k1_body(emb_ref, cl_ref, sums_ref, cnt_ref):
    i = pl.program_id(0)
    emb = emb_ref[...]
    nrm = jnp.sqrt(jnp.sum(emb * emb, axis=1, keepdims=True))
    embn = emb / (nrm + EPS)
    cl = cl_ref[...]
    iota_c = lax.broadcasted_iota(jnp.int32, (1, NCP), 1)
    onehot = (cl == iota_c).astype(jnp.float32)
    sums = lax.dot_general(onehot, embn, (((0,), (0,)), ((), ())),
                           preferred_element_type=jnp.float32, precision=lax.Precision.HIGHEST)
    ones = jnp.ones((BLK, 8), dtype=jnp.float32)
    cnts = lax.dot_general(onehot, ones, (((0,), (0,)), ((), ())),
                           preferred_element_type=jnp.float32)

    @pl.when(i == 0)
    def _():
        sums_ref[...] = jnp.zeros_like(sums_ref)
        cnt_ref[...] = jnp.zeros_like(cnt_ref)

    sums_ref[...] += sums
    cnt_ref[...] += cnts


def _run_k1(embeddings, clusters2d):
    grid = N // BLK
    return pl.pallas_call(
        _k1_body,
        grid=(grid,),
        in_specs=[
            pl.BlockSpec((BLK, EMB), lambda i: (i, 0)),
            pl.BlockSpec((BLK, 1), lambda i: (i, 0)),
        ],
        out_specs=[
            pl.BlockSpec((NCP, EMB), lambda i: (0, 0)),
            pl.BlockSpec((NCP, 8), lambda i: (0, 0)),
        ],
        out_shape=[
            jax.ShapeDtypeStruct((NCP, EMB), jnp.float32),
            jax.ShapeDtypeStruct((NCP, 8), jnp.float32),
        ],
    )(embeddings, clusters2d)


def _k2_body(sums_ref, cnt_ref, means_ref, idx_ref, wf_ref, ws_ref):
    sums = sums_ref[...]
    cnt = cnt_ref[:, 0:1]
    mean = sums / jnp.maximum(cnt, 1.0)
    nrm = jnp.sqrt(jnp.sum(mean * mean, axis=1, keepdims=True))
    means = mean / (nrm + EPS)
    means_ref[...] = means

    sim_d = lax.dot_general(means, means, (((1,), (1,)), ((), ())),
                            preferred_element_type=jnp.float32)
    sim = lax.dot_general(means, means, (((1,), (1,)), ((), ())),
                          preferred_element_type=jnp.float32, precision=lax.Precision.HIGHEST)
    iota_c = lax.broadcasted_iota(jnp.int32, (1, NCP), 1)
    row_i = lax.broadcasted_iota(jnp.int32, (NCP, 1), 0)
    valid_col = iota_c < NC
    work = jnp.where(valid_col, sim_d, NEG)
    mask, _, idxs, sels = _topk_iter(work, iota_c, K_SUP)

    y = jnp.clip(sim, -1 + 1e-7, 1 - 1e-7)
    s = jnp.sqrt((1 + y) / (1 - y))
    wmat = s / (1 + s)
    a = mask * wmat
    a = jnp.where(row_i < NC, a, 0.0)
    onescol = jnp.ones((NCP, 1), dtype=jnp.float32)
    colsum = _dot_exact_rhs(a, onescol, (((0,), (0,)), ((), ())))
    rowsum = _dot_exact_rhs(a, onescol, (((1,), (0,)), ((), ())))
    denom = colsum + rowsum

    for t in range(K_SUP):
        j = idxs[t]
        mx = jnp.sum(jnp.where(sels[t], sim, 0.0), axis=1, keepdims=True)
        yv = jnp.clip(mx, -1 + 1e-7, 1 - 1e-7)
        sv = jnp.sqrt((1 + yv) / (1 - yv))
        wv = sv / (1 + sv)
        oh = (iota_c == j).astype(jnp.float32)
        dg = _dot_exact_lhs(oh, denom, (((1,), (0,)), ((), ())))
        idx_ref[:, t:t + 1] = j
        wf_ref[:, t:t + 1] = wv / (dg + EPS)
        ws_ref[:, t:t + 1] = wv / (denom + EPS)


def _run_k2(sums, cnts):
    return pl.pallas_call(
        _k2_body,
        out_shape=[
            jax.ShapeDtypeStruct((NCP, EMB), jnp.float32),
            jax.ShapeDtypeStruct((NCP, K_SUP), jnp.int32),
            jax.ShapeDtypeStruct((NCP, K_SUP), jnp.float32),
            jax.ShapeDtypeStruct((NCP, K_SUP), jnp.float32),
        ],
    )(sums, cnts)


def _k3_body(emb_ref, nodes_ref, means_ref, sraw_ref, den_ref):
    i = pl.program_id(0)
    emb = emb_ref[...]
    nrm = jnp.sqrt(jnp.sum(emb * emb, axis=1, keepdims=True))
    embn = emb / (nrm + EPS)
    means = means_ref[...]
    bsim = lax.dot_general(embn, means, (((1,), (1,)), ((), ())),
                           preferred_element_type=jnp.float32)
    iota_c = lax.broadcasted_iota(jnp.int32, (1, NCP), 1)
    work = jnp.where(iota_c < NC, bsim, NEG)
    mask, _, _, _ = _topk_iter(work, iota_c, K_BIP)

    y = jnp.clip(bsim, -1 + 1e-7, 1 - 1e-7)
    w = mask * jnp.sqrt((1 + y) / (1 - y))

    nodes = nodes_ref[...]
    l1 = jnp.sum(jnp.abs(nodes), axis=1, keepdims=True)
    nodes_l1 = nodes / (l1 + EPS)

    sraw = _dot3(w, nodes_l1, (((0,), (0,)), ((), ())))
    ones = jnp.ones((BLK, 8), dtype=jnp.float32)
    den = _dot_exact_rhs(w, ones, (((0,), (0,)), ((), ())))

    @pl.when(i == 0)
    def _():
        sraw_ref[...] = jnp.zeros_like(sraw_ref)
        den_ref[...] = jnp.zeros_like(den_ref)

    sraw_ref[...] += sraw
    den_ref[...] += den


def _run_k3(embeddings, nodes, means):
    grid = N // BLK
    return pl.pallas_call(
        _k3_body,
        grid=(grid,),
        in_specs=[
            pl.BlockSpec((BLK, EMB), lambda i: (i, 0)),
            pl.BlockSpec((BLK, LAT), lambda i: (i, 0)),
            pl.BlockSpec((NCP, EMB), lambda i: (0, 0)),
        ],
        out_specs=[
            pl.BlockSpec((NCP, LAT), lambda i: (0, 0)),
            pl.BlockSpec((NCP, 8), lambda i: (0, 0)),
        ],
        out_shape=[
            jax.ShapeDtypeStruct((NCP, LAT), jnp.float32),
            jax.ShapeDtypeStruct((NCP, 8), jnp.float32),
        ],
    )(embeddings, nodes, means)


def _k4_body(sraw_ref, den_ref, means_ref, idxf_ref,
             wn1_ref, bn1_ref, wn2_ref, bn2_ref,
             w1at_ref, w1ab_ref, w1bt_ref, w1bb_ref, be1_ref,
             we2_ref, be2_ref,
             snenc_ref, sef_ref, ses_ref):
    sn = sraw_ref[...] / (den_ref[:, 0:1] + EPS)
    h = jax.nn.relu(_ln(jnp.dot(sn, wn1_ref[...],
                                preferred_element_type=jnp.float32)
                        + bn1_ref[...]))
    sn_enc = jax.nn.relu(_ln(jnp.dot(h, wn2_ref[...],
                                     preferred_element_type=jnp.float32)
                             + bn2_ref[...]))
    snenc_ref[...] = sn_enc

    means = means_ref[...]
    p = (jnp.dot(means, w1at_ref[...], preferred_element_type=jnp.float32)
         + jnp.dot(sn_enc, w1ab_ref[...], preferred_element_type=jnp.float32))
    q = (jnp.dot(means, w1bt_ref[...], preferred_element_type=jnp.float32)
         + jnp.dot(sn_enc, w1bb_ref[...], preferred_element_type=jnp.float32))

    idxf = idxf_ref[...]
    iota_c = lax.broadcasted_iota(jnp.int32, (1, NCP), 1)
    ohe = (idxf == iota_c).astype(jnp.float32)
    dims_nn = (((1,), (0,)), ((), ()))
    pg = _dot_exact_lhs(ohe, p, dims_nn)
    qg = _dot_exact_lhs(ohe, q, dims_nn)
    p_rep = jnp.reshape(jnp.broadcast_to(p[:, None, :], (NCP, K_SUP, HID)),
                        (NCP * K_SUP, HID))
    q_rep = jnp.reshape(jnp.broadcast_to(q[:, None, :], (NCP, K_SUP, HID)),
                        (NCP * K_SUP, HID))

    be1 = be1_ref[...]
    we2 = we2_ref[...]
    be2 = be2_ref[...]
    h1f = jax.nn.relu(_ln(p_rep + qg + be1))
    h1s = jax.nn.relu(_ln(pg + q_rep + be1))
    sef_ref[...] = jax.nn.relu(_ln(jnp.dot(h1f, we2,
                                           preferred_element_type=jnp.float32)
                                   + be2))
    ses_ref[...] = jax.nn.relu(_ln(jnp.dot(h1s, we2,
                                           preferred_element_type=jnp.float32)
                                   + be2))


def _run_k4(sraw, den, means, idxf, wn1, bn1, wn2, bn2,
            w1at, w1ab, w1bt, w1bb, be1, we2, be2):
    ne = NCP * K_SUP
    return pl.pallas_call(
        _k4_body,
        out_shape=[
            jax.ShapeDtypeStruct((NCP, LAT - EMB), jnp.float32),
            jax.ShapeDtypeStruct((ne, LAT), jnp.float32),
            jax.ShapeDtypeStruct((ne, LAT), jnp.float32),
        ],
    )(sraw, den, means, idxf, wn1, bn1, wn2, bn2,
      w1at, w1ab, w1bt, w1bb, be1, we2, be2)


@jax.jit
def kernel(x, embeddings, nodes, graph, clusters,
           Wn1, bn1, Wn2, bn2, We1, be1, We2, be2):
    del x, graph
    clusters2d = clusters.reshape(N, 1).astype(jnp.int32)
    sums, cnts = _run_k1(embeddings, clusters2d)
    means, idx, wf, ws = _run_k2(sums, cnts)
    sraw, den = _run_k3(embeddings, nodes, means)

    idxf = idx.reshape(NCP * K_SUP, 1)
    w1at, w1ab = We1[:EMB], We1[EMB:LAT]
    w1bt, w1bb = We1[LAT:LAT + EMB], We1[LAT + EMB:]
    sn_enc, sef, ses = _run_k4(
        sraw, den, means, idxf, Wn1, bn1.reshape(1, HID),
        Wn2, bn2.reshape(1, LAT - EMB),
        w1at, w1ab, w1bt, w1bb, be1.reshape(1, HID),
        We2, be2.reshape(1, LAT))

    supernodes = jnp.concatenate([means[:NC], sn_enc[:NC]], axis=-1)
    nedge = NC * K_SUP
    superedges = jnp.concatenate([sef[:nedge], ses[:nedge]], axis=0)
    sew = jnp.concatenate([wf.reshape(-1)[:nedge], ws.reshape(-1)[:nedge]])
    return supernodes, superedges, sew[:, None]


# speedup vs baseline: 8.4930x; 1.0817x over previous
import functools

import jax
import jax.numpy as jnp
from jax import lax
from jax.experimental import pallas as pl

N = 10000
LAT = 128
EMB = 16
NC = 500
NCP = 512
HID = 256
K_SUP = 8
K_BIP = 4
EPS = 1e-12
BLK = 1000
NEG = -3.0e38


def _bf16_split(x):
    hi = x.astype(jnp.bfloat16).astype(jnp.float32)
    return hi, x - hi


def _dot_exact_lhs(a, b, dims):
    b_hi, b_lo = _bf16_split(b)
    d1 = lax.dot_general(a, b_hi, dims, preferred_element_type=jnp.float32)
    d2 = lax.dot_general(a, b_lo, dims, preferred_element_type=jnp.float32)
    return d1 + d2


def _dot_exact_rhs(a, b, dims):
    a_hi, a_lo = _bf16_split(a)
    d1 = lax.dot_general(a_hi, b, dims, preferred_element_type=jnp.float32)
    d2 = lax.dot_general(a_lo, b, dims, preferred_element_type=jnp.float32)
    return d1 + d2


def _dot_exact_lhs3(a, b, dims):
    b1 = b.astype(jnp.bfloat16).astype(jnp.float32)
    r = b - b1
    b2 = r.astype(jnp.bfloat16).astype(jnp.float32)
    b3 = r - b2
    d1 = lax.dot_general(a, b1, dims, preferred_element_type=jnp.float32)
    d2 = lax.dot_general(a, b2, dims, preferred_element_type=jnp.float32)
    d3 = lax.dot_general(a, b3, dims, preferred_element_type=jnp.float32)
    return d1 + (d2 + d3)


def _dot3(a, b, dims):
    a_hi, a_lo = _bf16_split(a)
    b_hi, b_lo = _bf16_split(b)
    d1 = lax.dot_general(a_hi, b_hi, dims, preferred_element_type=jnp.float32)
    d2 = lax.dot_general(a_hi, b_lo, dims, preferred_element_type=jnp.float32)
    d3 = lax.dot_general(a_lo, b_hi, dims, preferred_element_type=jnp.float32)
    return d1 + (d2 + d3)


def _ln(h):
    m = h.mean(-1, keepdims=True)
    v = ((h - m) ** 2).mean(-1, keepdims=True)
    return (h - m) * lax.rsqrt(v + 1e-5)


def _topk_iter(work, iota_c, k):
    R, C = work.shape
    mask = jnp.zeros((R, C), dtype=jnp.float32)
    vals, idxs, sels = [], [], []
    for _ in range(k):
        mx = jnp.max(work, axis=1, keepdims=True)
        ismax = work == mx
        j = jnp.min(jnp.where(ismax, iota_c, C), axis=1, keepdims=True)
        sel = iota_c == j
        mask = jnp.where(sel, 1.0, mask)
        work = jnp.where(sel, NEG, work)
        vals.append(mx)
        idxs.append(j)
        sels.append(sel)
    return mask, vals, idxs, sels


def _k1_body(emb_ref, cl_ref, sums_ref, cnt_ref):
    i = pl.program_id(0)
    emb = emb_ref[...]
    nrm = jnp.sqrt(jnp.sum(emb * emb, axis=1, keepdims=True))
    embn = emb / (nrm + EPS)
    cl = cl_ref[...]
    iota_c = lax.broadcasted_iota(jnp.int32, (1, NCP), 1)
    onehot = (cl == iota_c).astype(jnp.float32)
    sums = _dot_exact_lhs3(onehot, embn, (((0,), (0,)), ((), ())))
    ones = jnp.ones((BLK, 8), dtype=jnp.float32)
    cnts = lax.dot_general(onehot, ones, (((0,), (0,)), ((), ())),
                           preferred_element_type=jnp.float32)

    @pl.when(i == 0)
    def _():
        sums_ref[...] = jnp.zeros_like(sums_ref)
        cnt_ref[...] = jnp.zeros_like(cnt_ref)

    sums_ref[...] += sums
    cnt_ref[...] += cnts


def _run_k1(embeddings, clusters2d):
    grid = N // BLK
    return pl.pallas_call(
        _k1_body,
        grid=(grid,),
        in_specs=[
            pl.BlockSpec((BLK, EMB), lambda i: (i, 0)),
            pl.BlockSpec((BLK, 1), lambda i: (i, 0)),
        ],
        out_specs=[
            pl.BlockSpec((NCP, EMB), lambda i: (0, 0)),
            pl.BlockSpec((NCP, 8), lambda i: (0, 0)),
        ],
        out_shape=[
            jax.ShapeDtypeStruct((NCP, EMB), jnp.float32),
            jax.ShapeDtypeStruct((NCP, 8), jnp.float32),
        ],
    )(embeddings, clusters2d)


def _k2_body(sums_ref, cnt_ref, means_ref, idx_ref, wf_ref, ws_ref):
    sums = sums_ref[...]
    cnt = cnt_ref[:, 0:1]
    mean = sums / jnp.maximum(cnt, 1.0)
    nrm = jnp.sqrt(jnp.sum(mean * mean, axis=1, keepdims=True))
    means = mean / (nrm + EPS)
    means_ref[...] = means

    sim_d = lax.dot_general(means, means, (((1,), (1,)), ((), ())),
                            preferred_element_type=jnp.float32)
    sim = lax.dot_general(means, means, (((1,), (1,)), ((), ())),
                          preferred_element_type=jnp.float32, precision=lax.Precision.HIGHEST)
    iota_c = lax.broadcasted_iota(jnp.int32, (1, NCP), 1)
    row_i = lax.broadcasted_iota(jnp.int32, (NCP, 1), 0)
    valid_col = iota_c < NC
    work = jnp.where(valid_col, sim_d, NEG)
    mask, _, idxs, sels = _topk_iter(work, iota_c, K_SUP)

    y = jnp.clip(sim, -1 + 1e-7, 1 - 1e-7)
    s = jnp.sqrt((1 + y) / (1 - y))
    wmat = s / (1 + s)
    a = mask * wmat
    a = jnp.where(row_i < NC, a, 0.0)
    onescol = jnp.ones((NCP, 1), dtype=jnp.float32)
    colsum = _dot_exact_rhs(a, onescol, (((0,), (0,)), ((), ())))
    rowsum = _dot_exact_rhs(a, onescol, (((1,), (0,)), ((), ())))
    denom = colsum + rowsum

    for t in range(K_SUP):
        j = idxs[t]
        mx = jnp.sum(jnp.where(sels[t], sim, 0.0), axis=1, keepdims=True)
        yv = jnp.clip(mx, -1 + 1e-7, 1 - 1e-7)
        sv = jnp.sqrt((1 + yv) / (1 - yv))
        wv = sv / (1 + sv)
        oh = (iota_c == j).astype(jnp.float32)
        dg = _dot_exact_lhs(oh, denom, (((1,), (0,)), ((), ())))
        idx_ref[:, t:t + 1] = j
        wf_ref[:, t:t + 1] = wv / (dg + EPS)
        ws_ref[:, t:t + 1] = wv / (denom + EPS)


def _run_k2(sums, cnts):
    return pl.pallas_call(
        _k2_body,
        out_shape=[
            jax.ShapeDtypeStruct((NCP, EMB), jnp.float32),
            jax.ShapeDtypeStruct((NCP, K_SUP), jnp.int32),
            jax.ShapeDtypeStruct((NCP, K_SUP), jnp.float32),
            jax.ShapeDtypeStruct((NCP, K_SUP), jnp.float32),
        ],
    )(sums, cnts)


def _k3_body(emb_ref, nodes_ref, means_ref, sraw_ref, den_ref):
    i = pl.program_id(0)
    emb = emb_ref[...]
    nrm = jnp.sqrt(jnp.sum(emb * emb, axis=1, keepdims=True))
    embn = emb / (nrm + EPS)
    means = means_ref[...]
    bsim = lax.dot_general(embn, means, (((1,), (1,)), ((), ())),
                           preferred_element_type=jnp.float32)
    iota_c = lax.broadcasted_iota(jnp.int32, (1, NCP), 1)
    work = jnp.where(iota_c < NC, bsim, NEG)
    mask, _, _, _ = _topk_iter(work, iota_c, K_BIP)

    y = jnp.clip(bsim, -1 + 1e-7, 1 - 1e-7)
    w = mask * jnp.sqrt((1 + y) / (1 - y))

    nodes = nodes_ref[...]
    l1 = jnp.sum(jnp.abs(nodes), axis=1, keepdims=True)
    nodes_l1 = nodes / (l1 + EPS)

    sraw = _dot3(w, nodes_l1, (((0,), (0,)), ((), ())))
    ones = jnp.ones((BLK, 8), dtype=jnp.float32)
    den = _dot_exact_rhs(w, ones, (((0,), (0,)), ((), ())))

    @pl.when(i == 0)
    def _():
        sraw_ref[...] = jnp.zeros_like(sraw_ref)
        den_ref[...] = jnp.zeros_like(den_ref)

    sraw_ref[...] += sraw
    den_ref[...] += den


def _run_k3(embeddings, nodes, means):
    grid = N // BLK
    return pl.pallas_call(
        _k3_body,
        grid=(grid,),
        in_specs=[
            pl.BlockSpec((BLK, EMB), lambda i: (i, 0)),
            pl.BlockSpec((BLK, LAT), lambda i: (i, 0)),
            pl.BlockSpec((NCP, EMB), lambda i: (0, 0)),
        ],
        out_specs=[
            pl.BlockSpec((NCP, LAT), lambda i: (0, 0)),
            pl.BlockSpec((NCP, 8), lambda i: (0, 0)),
        ],
        out_shape=[
            jax.ShapeDtypeStruct((NCP, LAT), jnp.float32),
            jax.ShapeDtypeStruct((NCP, 8), jnp.float32),
        ],
    )(embeddings, nodes, means)


def _k4_body(sraw_ref, den_ref, means_ref, idxf_ref,
             wn1_ref, bn1_ref, wn2_ref, bn2_ref,
             w1at_ref, w1ab_ref, w1bt_ref, w1bb_ref, be1_ref,
             we2_ref, be2_ref,
             snenc_ref, se_ref):
    sn = sraw_ref[...] / (den_ref[:, 0:1] + EPS)
    h = jax.nn.relu(_ln(jnp.dot(sn, wn1_ref[...],
                                preferred_element_type=jnp.float32)
                        + bn1_ref[...]))
    sn_enc = jax.nn.relu(_ln(jnp.dot(h, wn2_ref[...],
                                     preferred_element_type=jnp.float32)
                             + bn2_ref[...]))
    snenc_ref[...] = sn_enc

    means = means_ref[...]
    p = (jnp.dot(means, w1at_ref[...], preferred_element_type=jnp.float32)
         + jnp.dot(sn_enc, w1ab_ref[...], preferred_element_type=jnp.float32))
    q = (jnp.dot(means, w1bt_ref[...], preferred_element_type=jnp.float32)
         + jnp.dot(sn_enc, w1bb_ref[...], preferred_element_type=jnp.float32))

    idxf = idxf_ref[...]
    iota_c = lax.broadcasted_iota(jnp.int32, (1, NCP), 1)
    ohe = (idxf == iota_c).astype(jnp.float32)
    dims_nn = (((1,), (0,)), ((), ()))
    pg = _dot_exact_lhs(ohe, p, dims_nn)
    qg = _dot_exact_lhs(ohe, q, dims_nn)
    p_rep = jnp.reshape(jnp.broadcast_to(p[:, None, :], (NCP, K_SUP, HID)),
                        (NCP * K_SUP, HID))
    q_rep = jnp.reshape(jnp.broadcast_to(q[:, None, :], (NCP, K_SUP, HID)),
                        (NCP * K_SUP, HID))

    be1 = be1_ref[...]
    we2 = we2_ref[...]
    be2 = be2_ref[...]
    h1f = jax.nn.relu(_ln(p_rep + qg + be1))
    h1s = jax.nn.relu(_ln(pg + q_rep + be1))
    x2f = jax.nn.relu(_ln(jnp.dot(h1f, we2,
                                  preferred_element_type=jnp.float32)
                          + be2))
    x2s = jax.nn.relu(_ln(jnp.dot(h1s, we2,
                                  preferred_element_type=jnp.float32)
                          + be2))
    nv = NC * K_SUP
    se_ref[0:nv, :] = x2f[0:nv, :]
    se_ref[nv:2 * nv, :] = x2s[0:nv, :]


def _run_k4(sraw, den, means, idxf, wn1, bn1, wn2, bn2,
            w1at, w1ab, w1bt, w1bb, be1, we2, be2):
    return pl.pallas_call(
        _k4_body,
        out_shape=[
            jax.ShapeDtypeStruct((NCP, LAT - EMB), jnp.float32),
            jax.ShapeDtypeStruct((2 * NC * K_SUP, LAT), jnp.float32),
        ],
    )(sraw, den, means, idxf, wn1, bn1, wn2, bn2,
      w1at, w1ab, w1bt, w1bb, be1, we2, be2)


@jax.jit
def kernel(x, embeddings, nodes, graph, clusters,
           Wn1, bn1, Wn2, bn2, We1, be1, We2, be2):
    del x, graph
    clusters2d = clusters.reshape(N, 1).astype(jnp.int32)
    sums, cnts = _run_k1(embeddings, clusters2d)
    means, idx, wf, ws = _run_k2(sums, cnts)
    sraw, den = _run_k3(embeddings, nodes, means)

    idxf = idx.reshape(NCP * K_SUP, 1)
    w1at, w1ab = We1[:EMB], We1[EMB:LAT]
    w1bt, w1bb = We1[LAT:LAT + EMB], We1[LAT + EMB:]
    sn_enc, superedges = _run_k4(
        sraw, den, means, idxf, Wn1, bn1.reshape(1, HID),
        Wn2, bn2.reshape(1, LAT - EMB),
        w1at, w1ab, w1bt, w1bb, be1.reshape(1, HID),
        We2, be2.reshape(1, LAT))

    supernodes = jnp.concatenate([means[:NC], sn_enc[:NC]], axis=-1)
    nedge = NC * K_SUP
    sew = jnp.concatenate([wf.reshape(-1)[:nedge], ws.reshape(-1)[:nedge]])
    return supernodes, superedges, sew[:, None]
